# Initial kernel scaffold; baseline (speedup 1.0000x reference)
import jax
import jax.numpy as jnp
from jax.experimental import pallas as pl


def kernel(seq_len, table):
    raise NotImplementedError("write your pallas kernel here")


# speedup vs baseline: 1.4734x; 1.4734x over previous
import functools

import jax
import jax.numpy as jnp
from jax import lax
from jax.experimental import pallas as pl
from jax.experimental.pallas import tpu as pltpu
from jax.experimental.pallas import tpu_sc as plsc

_L = 16


@functools.lru_cache(maxsize=None)
def _make_gather(n_rows: int, d_model: int):
    info = plsc.get_sparse_core_info()
    nw = info.num_cores * info.num_subcores
    rows_per_w = n_rows // nw
    r = 64
    while rows_per_w % r:
        r //= 2
    n_chunks = rows_per_w // r

    mesh = plsc.VectorSubcoreMesh(core_axis_name="c", subcore_axis_name="s")

    @functools.partial(
        pl.kernel,
        mesh=mesh,
        out_type=jax.ShapeDtypeStruct((n_rows, d_model), jnp.float32),
        scratch_types=[
            pltpu.VMEM((_L,), jnp.int32),
            pltpu.VMEM((r,), jnp.int32),
            pltpu.VMEM((r, d_model), jnp.float32),
            pltpu.SemaphoreType.DMA,
        ],
    )
    def k(seq_hbm, table_hbm, out_hbm, seq_v, idx_v, rows_v, sem):
        wid = lax.axis_index("s") * info.num_cores + lax.axis_index("c")
        base = wid * rows_per_w
        pltpu.sync_copy(seq_hbm, seq_v)
        sl = seq_v[...]
        for c in range(n_chunks):
            row0 = base + c * r
            for j in range(r // _L):
                idx_v[pl.ds(j * _L, _L)] = lax.rem(
                    (row0 + j * _L) + lax.iota(jnp.int32, _L), sl)
            pltpu.async_copy(table_hbm.at[idx_v], rows_v, sem).wait()
            pltpu.sync_copy(rows_v, out_hbm.at[pl.ds(row0, r)])

    return k


def kernel(seq_len, table):
    n_rows, d_model = table.shape
    seq_arr = jnp.full((_L,), seq_len, dtype=jnp.int32)
    return _make_gather(n_rows, d_model)(seq_arr, table)


# speedup vs baseline: 1.5440x; 1.0479x over previous
import functools

import jax
import jax.numpy as jnp
from jax import lax
from jax.experimental import pallas as pl
from jax.experimental.pallas import tpu as pltpu
from jax.experimental.pallas import tpu_sc as plsc

_L = 16


@functools.lru_cache(maxsize=None)
def _make_gather(n_rows: int, d_model: int):
    info = plsc.get_sparse_core_info()
    nw = info.num_cores * info.num_subcores
    rows_per_w = n_rows // nw
    r = 32
    while rows_per_w % r:
        r //= 2
    n_chunks = rows_per_w // r

    mesh = plsc.VectorSubcoreMesh(core_axis_name="c", subcore_axis_name="s")

    @functools.partial(
        pl.kernel,
        mesh=mesh,
        out_type=jax.ShapeDtypeStruct((n_rows, d_model), jnp.float32),
        scratch_types=[
            pltpu.VMEM((_L,), jnp.int32),
            pltpu.VMEM((2, r), jnp.int32),
            pltpu.VMEM((2, r, d_model), jnp.float32),
            pltpu.SemaphoreType.DMA,
        ],
    )
    def k(seq_hbm, table_hbm, out_hbm, seq_v, idx_v, rows_v, sem):
        wid = lax.axis_index("s") * info.num_cores + lax.axis_index("c")
        base = wid * rows_per_w
        pltpu.sync_copy(seq_hbm, seq_v)
        sl = seq_v[...]

        def start_gather(c):
            b = c % 2
            row0 = base + c * r
            for j in range(r // _L):
                idx_v[b, pl.ds(j * _L, _L)] = lax.rem(
                    (row0 + j * _L) + lax.iota(jnp.int32, _L), sl)
            return pltpu.async_copy(table_hbm.at[idx_v.at[b]],
                                    rows_v.at[b], sem)

        g = start_gather(0)
        for c in range(n_chunks):
            g_next = start_gather(c + 1) if c + 1 < n_chunks else None
            g.wait()
            pltpu.sync_copy(rows_v.at[c % 2], out_hbm.at[pl.ds(base + c * r, r)])
            g = g_next

    return k


def kernel(seq_len, table):
    n_rows, d_model = table.shape
    seq_arr = jnp.full((_L,), seq_len, dtype=jnp.int32)
    return _make_gather(n_rows, d_model)(seq_arr, table)


# speedup vs baseline: 1.5440x; 1.0000x over previous
import functools

import jax
import jax.numpy as jnp
from jax import lax
from jax.experimental import pallas as pl
from jax.experimental.pallas import tpu as pltpu
from jax.experimental.pallas import tpu_sc as plsc

_L = 16


@functools.lru_cache(maxsize=None)
def _make_gather(n_rows: int, d_model: int):
    info = plsc.get_sparse_core_info()
    nw = info.num_cores * info.num_subcores
    rows_per_w = n_rows // nw
    r = 32
    while rows_per_w % r:
        r //= 2
    n_chunks = rows_per_w // r

    mesh = plsc.VectorSubcoreMesh(core_axis_name="c", subcore_axis_name="s")

    @functools.partial(
        pl.kernel,
        mesh=mesh,
        out_type=jax.ShapeDtypeStruct((n_rows, d_model), jnp.float32),
        scratch_types=[
            pltpu.VMEM((_L,), jnp.int32),
            pltpu.VMEM((r,), jnp.int32),
            pltpu.VMEM((r,), jnp.int32),
            pltpu.VMEM((2, r, d_model), jnp.float32),
            pltpu.SemaphoreType.DMA,
        ],
    )
    def k(seq_hbm, table_hbm, out_hbm, seq_v, idx0_v, idx1_v, rows_v, sem):
        wid = lax.axis_index("s") * info.num_cores + lax.axis_index("c")
        base = wid * rows_per_w
        pltpu.sync_copy(seq_hbm, seq_v)
        sl = seq_v[...]

        def start_gather(c):
            b = c % 2
            idx_v = idx0_v if b == 0 else idx1_v
            row0 = base + c * r
            for j in range(r // _L):
                idx_v[pl.ds(j * _L, _L)] = lax.rem(
                    (row0 + j * _L) + lax.iota(jnp.int32, _L), sl)
            return pltpu.async_copy(table_hbm.at[idx_v],
                                    rows_v.at[b], sem)

        g = start_gather(0)
        for c in range(n_chunks):
            g_next = start_gather(c + 1) if c + 1 < n_chunks else None
            g.wait()
            pltpu.sync_copy(rows_v.at[c % 2], out_hbm.at[pl.ds(base + c * r, r)])
            g = g_next

    return k


def kernel(seq_len, table):
    n_rows, d_model = table.shape
    seq_arr = jnp.full((_L,), seq_len, dtype=jnp.int32)
    return _make_gather(n_rows, d_model)(seq_arr, table)
